# baseline (speedup 1.0000x reference)
import functools

import jax
import jax.numpy as jnp
import numpy as np
from jax import lax
from jax.experimental import pallas as pl
from jax.experimental.pallas import tpu as pltpu
from jax.experimental.pallas import tpu_sc as plsc

NLV = 16
FEAT = 2
LOG2T = 19
TBL = 1 << LOG2T
BASEG = 16
FINEST = 512
NPTS = 524288
_SCALE = np.exp(np.log(FINEST / BASEG) / (NLV - 1))
RESL = [int(np.floor(BASEG * _SCALE ** l)) for l in range(NLV)]
P1 = 2654435761 - (1 << 32)
P2 = 805459861

NSC = 2
NSUB = 16
NW = NSC * NSUB
PPW = NPTS // NW
CH = 64
GRP = CH // 16
NCHUNK = PPW // CH
ODIM = NLV * FEAT
PK = 4
PKW = PK * FEAT

RES_LVLS = [0, 1, 2]
STREAM_LVLS = [l for l in range(NLV) if l not in RES_LVLS]
NSTR = len(STREAM_LVLS)


def _round32(v):
    return (v + 31) // 32 * 32


_OFF = []
_acc = 0
for _l in RES_LVLS:
    _OFF.append(_acc)
    _acc = _round32(_acc + (RESL[_l] + 1) ** 3)
TABV_ROWS = _acc // PK


def _pos_int(xv, o, d, res, xo=0):
    px = xv[pl.ds(xo + d * CH + o, 16)]
    pos = px * jnp.float32(res)
    pi = pos.astype(jnp.int32)
    return pos, pi


def _weights(xv, o, res, xo=0):
    w = []
    pis = []
    for d in range(3):
        pos, pi = _pos_int(xv, o, d, res, xo)
        fr = pos - pi.astype(jnp.float32)
        w.append((jnp.float32(1.0) - fr, fr))
        pis.append(pi)
    wxy = {}
    for by in range(2):
        for bx in range(2):
            wxy[(bx, by)] = w[0][bx] * w[1][by]
    return wxy, w[2], pis


def _corner_indices(l, pi):
    res = RESL[l]
    s = res + 1
    out = []
    if s ** 3 <= TBL:
        base = pi[0] + pi[1] * s + pi[2] * (s * s) + l * TBL
        for c8 in range(8):
            k = (c8 & 1) + ((c8 >> 1) & 1) * s + ((c8 >> 2) & 1) * (s * s)
            out.append(base + k)
    else:
        h0a = pi[0]
        h0b = pi[0] + 1
        h1a = pi[1] * P1
        h1b = h1a + P1
        h2a = pi[2] * P2
        h2b = h2a + P2
        for c8 in range(8):
            hx = h0b if (c8 & 1) else h0a
            hy = h1b if (c8 & 2) else h1a
            hz = h2b if (c8 & 4) else h2a
            out.append(((hx ^ hy ^ hz) & (TBL - 1)) + l * TBL)
    return out


def _emit_idx_level(l, si, xv, idxv, xo=0):
    res = RESL[l]

    @pl.loop(0, GRP, unroll=2)
    def _g(g):
        o = g * 16
        pi = [_pos_int(xv, o, d, res, xo)[1] for d in range(3)]
        for c8, idx in enumerate(_corner_indices(l, pi)):
            row = idxv.at[si]
            row[pl.ds(c8 * CH + o, 16)] = idx >> 2

def _lerp8(corner_idx, rows_of, featv, wxy, wz):
    acc0 = acc1 = None
    for c8 in range(8):
        w = wxy[(c8 & 1, (c8 >> 1) & 1)] * wz[(c8 >> 2) & 1]
        col0 = (corner_idx[c8] & 3) * 2
        rows = rows_of(c8)
        f0 = plsc.load_gather(featv, [rows, col0])
        f1 = plsc.load_gather(featv, [rows, col0 + 1])
        if acc0 is None:
            acc0, acc1 = w * f0, w * f1
        else:
            acc0, acc1 = acc0 + w * f0, acc1 + w * f1
    return acc0, acc1


def _out_scatter(l, outv, iota, o, acc0, acc1):
    cols = o + iota
    z = iota * 0
    plsc.store_scatter(outv, [z + (2 * l), cols], acc0)
    plsc.store_scatter(outv, [z + (2 * l + 1), cols], acc1)


def _emit_trilerp_level(l, si, xv, featv, outv, iota, xo=0):
    res = RESL[l]

    @pl.loop(0, GRP, unroll=2)
    def _g(g):
        o = g * 16
        wxy, wz, pi = _weights(xv, o, res, xo)
        cidx = _corner_indices(l, pi)
        rowg = o + iota
        acc0, acc1 = _lerp8(
            cidx, lambda c8: rowg + (si * 8 + c8) * CH, featv, wxy, wz)
        _out_scatter(l, outv, iota, o, acc0, acc1)


def _emit_resident_level(l, off, xv, tabv, outv, iota, xo=0):
    res = RESL[l]

    @pl.loop(0, GRP)
    def _g(g):
        o = g * 16
        wxy, wz, pi = _weights(xv, o, res, xo)
        cidx = [c - l * TBL + off for c in _corner_indices(l, pi)]
        acc0, acc1 = _lerp8(
            cidx, lambda c8: cidx[c8] >> 2, tabv, wxy, wz)
        _out_scatter(l, outv, iota, o, acc0, acc1)


def _body(xt_ref, tab_ref, out_ref, tabv, xv, idxv, featv, outv, sem, osem,
          xsem):
    cid = lax.axis_index("c")
    sid = lax.axis_index("s")
    wid = sid * NSC + cid
    iota = lax.iota(jnp.int32, 16)

    for i, l in enumerate(RES_LVLS):
        nr = _round32((RESL[l] + 1) ** 3) // PK
        pltpu.sync_copy(tab_ref.at[pl.ds(l * (TBL // PK), nr)],
                        tabv.at[pl.ds(_OFF[i] // PK, nr)])

    base_pt = wid * PPW
    XB = 3 * CH

    def _load_x(pt, xo):
        for d in range(3):
            pltpu.sync_copy(xt_ref.at[pl.ds(d * NPTS + pt, CH)],
                            xv.at[pl.ds(xo + d * CH, CH)])

    def _load_x_async(pt, xo):
        for d in range(3):
            pltpu.async_copy(xt_ref.at[pl.ds(d * NPTS + pt, CH)],
                             xv.at[pl.ds(xo + d * CH, CH)], xsem)

    def _wait_x():
        for d in range(3):
            pltpu.make_async_copy(xt_ref.at[pl.ds(0, CH)],
                                  xv.at[pl.ds(d * CH, CH)], xsem).wait()

    def _fire(si):
        return pltpu.async_copy(
            tab_ref.at[idxv.at[si]],
            featv.at[pl.ds(si * 8 * CH, 8 * CH)], sem)

    _load_x(base_pt, 0)
    for si, l in enumerate(STREAM_LVLS):
        _emit_idx_level(l, si, xv, idxv, 0)
        _fire(si)

    @pl.loop(0, NCHUNK)
    def _chunk(ci):
        pt0 = base_pt + ci * CH
        cur = (ci % 2) * XB
        nxt = XB - cur
        nb = pt0 // 128
        nl0 = (ci % 2) * CH

        @pl.when(ci < NCHUNK - 1)
        def _prefetch_x():
            _load_x_async(pt0 + CH, nxt)

        @pl.when(ci > 0)
        def _drain_out():
            for fb in range(4):
                pltpu.make_async_copy(
                    outv.at[pl.ds(fb * 8, 8)],
                    out_ref.at[fb, nb, :, pl.ds(nl0, CH)], osem).wait()

        for i, l in enumerate(RES_LVLS):
            _emit_resident_level(l, _OFF[i], xv, tabv, outv, iota, cur)
        for si, l in enumerate(STREAM_LVLS):
            pltpu.make_async_copy(
                tab_ref.at[idxv.at[si]],
                featv.at[pl.ds(si * 8 * CH, 8 * CH)], sem).wait()
            _emit_trilerp_level(l, si, xv, featv, outv, iota, cur)

            @pl.when(ci < NCHUNK - 1)
            def _refill():
                if si == 0:
                    _wait_x()
                _emit_idx_level(l, si, xv, idxv, nxt)
                _fire(si)

        for fb in range(4):
            pltpu.async_copy(outv.at[pl.ds(fb * 8, 8)],
                             out_ref.at[fb, nb, :, pl.ds(nl0, CH)], osem)
    for fb in range(4):
        pltpu.make_async_copy(
            outv.at[pl.ds(fb * 8, 8)],
            out_ref.at[0, 0, :, pl.ds(0, CH)], osem).wait()


_MESH = plsc.VectorSubcoreMesh(
    core_axis_name="c", subcore_axis_name="s", num_cores=NSC,
    num_subcores=NSUB)
_CPARAMS = pltpu.CompilerParams(
    needs_layout_passes=False, use_tc_tiling_on_sc=False)

_encode = functools.partial(
    pl.kernel,
    out_type=jax.ShapeDtypeStruct((4, NPTS // 128, 8, 128), jnp.float32),
    mesh=_MESH,
    compiler_params=_CPARAMS,
    scratch_types=[
        pltpu.VMEM((TABV_ROWS, PKW), jnp.float32),
        pltpu.VMEM((2 * 3 * CH,), jnp.float32),
        pltpu.VMEM((NSTR, 8 * CH), jnp.int32),
        pltpu.VMEM((NSTR * 8 * CH, PKW), jnp.float32),
        pltpu.VMEM((ODIM, CH), jnp.float32),
        pltpu.SemaphoreType.DMA,
        pltpu.SemaphoreType.DMA,
        pltpu.SemaphoreType.DMA,
    ],
)(_body)


REP_IN_ROWS = NLV * (TBL // 128) * FEAT
REP_CH = 128
REP_PER_W = REP_IN_ROWS // NW


def _repack_body(tn_ref, pk_ref, inb, outb, sem):
    cid = lax.axis_index("c")
    sid = lax.axis_index("s")
    wid = sid * NSC + cid
    iota = lax.iota(jnp.int32, 16)
    r0 = wid * REP_PER_W

    @pl.loop(0, REP_PER_W // REP_CH)
    def _ck(ck):
        rin = r0 + ck * REP_CH
        pltpu.sync_copy(tn_ref.at[pl.ds(rin, REP_CH)], inb)

        @pl.loop(0, REP_CH // 2)
        def _pair(k):
            for g in range(8):
                o = g * 16
                f0 = inb[2 * k, pl.ds(o, 16)]
                f1 = inb[2 * k + 1, pl.ds(o, 16)]
                q0 = 2 * o + 2 * iota
                rows0 = k * 32 + (q0 >> 3)
                plsc.store_scatter(outb, [rows0, q0 & 7], f0)
                q1 = q0 + 1
                plsc.store_scatter(outb, [k * 32 + (q1 >> 3), q1 & 7], f1)

        pltpu.sync_copy(outb, pk_ref.at[pl.ds((rin // 2) * 32, REP_CH * 16)])


_repack = functools.partial(
    pl.kernel,
    out_type=jax.ShapeDtypeStruct((NLV * TBL // PK, PKW), jnp.float32),
    mesh=_MESH,
    compiler_params=_CPARAMS,
    scratch_types=[
        pltpu.VMEM((REP_CH, 128), jnp.float32),
        pltpu.VMEM((REP_CH * 16, PKW), jnp.float32),
        pltpu.SemaphoreType.DMA,
    ],
)(_repack_body)


def kernel(x, table):
    xt = x.T.reshape(-1)
    tabn = (table.reshape(NLV, TBL // 128, 128, FEAT)
            .transpose(0, 1, 3, 2)
            .reshape(REP_IN_ROWS, 128))
    tab4 = _repack(tabn)
    out4 = _encode(xt, tab4)
    return out4.transpose(1, 3, 0, 2).reshape(NPTS, ODIM)


# speedup vs baseline: 1.0110x; 1.0110x over previous
import functools

import jax
import jax.numpy as jnp
import numpy as np
from jax import lax
from jax.experimental import pallas as pl
from jax.experimental.pallas import tpu as pltpu
from jax.experimental.pallas import tpu_sc as plsc

NLV = 16
FEAT = 2
LOG2T = 19
TBL = 1 << LOG2T
BASEG = 16
FINEST = 512
NPTS = 524288
_SCALE = np.exp(np.log(FINEST / BASEG) / (NLV - 1))
RESL = [int(np.floor(BASEG * _SCALE ** l)) for l in range(NLV)]
P1 = 2654435761 - (1 << 32)
P2 = 805459861

NSC = 2
NSUB = 16
NW = NSC * NSUB
PPW = NPTS // NW
CH = 64
GRP = CH // 16
NCHUNK = PPW // CH
ODIM = NLV * FEAT
PK = 4
PKW = PK * FEAT

RES_LVLS = [0, 1, 2]
STREAM_LVLS = [l for l in range(NLV) if l not in RES_LVLS]
NSTR = len(STREAM_LVLS)


def _round32(v):
    return (v + 31) // 32 * 32


_OFF = []
_acc = 0
for _l in RES_LVLS:
    _OFF.append(_acc)
    _acc = _round32(_acc + (RESL[_l] + 1) ** 3)
TABV_ROWS = _acc // PK


def _pos_int(xv, o, d, res, xo=0):
    px = xv[pl.ds(xo + d * CH + o, 16)]
    pos = px * jnp.float32(res)
    pi = pos.astype(jnp.int32)
    return pos, pi


def _weights(xv, o, res, xo=0):
    w = []
    pis = []
    for d in range(3):
        pos, pi = _pos_int(xv, o, d, res, xo)
        fr = pos - pi.astype(jnp.float32)
        w.append((jnp.float32(1.0) - fr, fr))
        pis.append(pi)
    wxy = {}
    for by in range(2):
        for bx in range(2):
            wxy[(bx, by)] = w[0][bx] * w[1][by]
    return wxy, w[2], pis


def _corner_indices(l, pi):
    res = RESL[l]
    s = res + 1
    out = []
    if s ** 3 <= TBL:
        base = pi[0] + pi[1] * s + pi[2] * (s * s) + l * TBL
        for c8 in range(8):
            k = (c8 & 1) + ((c8 >> 1) & 1) * s + ((c8 >> 2) & 1) * (s * s)
            out.append(base + k)
    else:
        h0a = pi[0]
        h0b = pi[0] + 1
        h1a = pi[1] * P1
        h1b = h1a + P1
        h2a = pi[2] * P2
        h2b = h2a + P2
        for c8 in range(8):
            hx = h0b if (c8 & 1) else h0a
            hy = h1b if (c8 & 2) else h1a
            hz = h2b if (c8 & 4) else h2a
            out.append(((hx ^ hy ^ hz) & (TBL - 1)) + l * TBL)
    return out


def _emit_idx_level(l, si, xv, idxv, xo=0):
    res = RESL[l]

    @pl.loop(0, GRP, unroll=2)
    def _g(g):
        o = g * 16
        pi = [_pos_int(xv, o, d, res, xo)[1] for d in range(3)]
        for c8, idx in enumerate(_corner_indices(l, pi)):
            row = idxv.at[si]
            row[pl.ds(c8 * CH + o, 16)] = idx >> 2

def _lerp8(corner_idx, rows_of, featv, wxy, wz):
    acc0 = acc1 = None
    for c8 in range(8):
        w = wxy[(c8 & 1, (c8 >> 1) & 1)] * wz[(c8 >> 2) & 1]
        col0 = (corner_idx[c8] & 3) * 2
        rows = rows_of(c8)
        f0 = plsc.load_gather(featv, [rows, col0])
        f1 = plsc.load_gather(featv, [rows, col0 + 1])
        if acc0 is None:
            acc0, acc1 = w * f0, w * f1
        else:
            acc0, acc1 = acc0 + w * f0, acc1 + w * f1
    return acc0, acc1


def _out_scatter(l, outv, iota, o, acc0, acc1):
    cols = o + iota
    z = iota * 0
    plsc.store_scatter(outv, [z + (2 * l), cols], acc0)
    plsc.store_scatter(outv, [z + (2 * l + 1), cols], acc1)


def _emit_trilerp_level(l, si, xv, featv, outv, iota, xo=0):
    res = RESL[l]

    @pl.loop(0, GRP, unroll=2)
    def _g(g):
        o = g * 16
        wxy, wz, pi = _weights(xv, o, res, xo)
        cidx = _corner_indices(l, pi)
        rowg = o + iota
        acc0, acc1 = _lerp8(
            cidx, lambda c8: rowg + (si * 8 + c8) * CH, featv, wxy, wz)
        _out_scatter(l, outv, iota, o, acc0, acc1)


def _emit_resident_level(l, off, xv, tabv, outv, iota, xo=0):
    res = RESL[l]

    @pl.loop(0, GRP)
    def _g(g):
        o = g * 16
        wxy, wz, pi = _weights(xv, o, res, xo)
        cidx = [c - l * TBL + off for c in _corner_indices(l, pi)]
        acc0, acc1 = _lerp8(
            cidx, lambda c8: cidx[c8] >> 2, tabv, wxy, wz)
        _out_scatter(l, outv, iota, o, acc0, acc1)


def _body(xt_ref, tab_ref, out_ref, tabv, xv, idxv, featv, outv, sem, osem):
    cid = lax.axis_index("c")
    sid = lax.axis_index("s")
    wid = sid * NSC + cid
    iota = lax.iota(jnp.int32, 16)

    for i, l in enumerate(RES_LVLS):
        nr = _round32((RESL[l] + 1) ** 3) // PK
        pltpu.sync_copy(tab_ref.at[pl.ds(l * (TBL // PK), nr)],
                        tabv.at[pl.ds(_OFF[i] // PK, nr)])

    base_pt = wid * PPW
    XB = 3 * CH

    def _load_x(pt, xo):
        for d in range(3):
            pltpu.sync_copy(xt_ref.at[pl.ds(d * NPTS + pt, CH)],
                            xv.at[pl.ds(xo + d * CH, CH)])


    def _fire(si):
        return pltpu.async_copy(
            tab_ref.at[idxv.at[si]],
            featv.at[pl.ds(si * 8 * CH, 8 * CH)], sem)

    _load_x(base_pt, 0)
    for si, l in enumerate(STREAM_LVLS):
        _emit_idx_level(l, si, xv, idxv, 0)
        _fire(si)

    @pl.loop(0, NCHUNK)
    def _chunk(ci):
        pt0 = base_pt + ci * CH
        cur = (ci % 2) * XB
        nxt = XB - cur
        nb = pt0 // 128
        nl0 = (ci % 2) * CH

        @pl.when(ci < NCHUNK - 1)
        def _prefetch_x():
            _load_x(pt0 + CH, nxt)

        @pl.when(ci > 0)
        def _drain_out():
            for fb in range(4):
                pltpu.make_async_copy(
                    outv.at[pl.ds(fb * 8, 8)],
                    out_ref.at[fb, nb, :, pl.ds(nl0, CH)], osem).wait()

        for i, l in enumerate(RES_LVLS):
            _emit_resident_level(l, _OFF[i], xv, tabv, outv, iota, cur)
        for si, l in enumerate(STREAM_LVLS):
            pltpu.make_async_copy(
                tab_ref.at[idxv.at[si]],
                featv.at[pl.ds(si * 8 * CH, 8 * CH)], sem).wait()
            _emit_trilerp_level(l, si, xv, featv, outv, iota, cur)

            @pl.when(ci < NCHUNK - 1)
            def _refill():
                _emit_idx_level(l, si, xv, idxv, nxt)
                _fire(si)

        for fb in range(4):
            pltpu.async_copy(outv.at[pl.ds(fb * 8, 8)],
                             out_ref.at[fb, nb, :, pl.ds(nl0, CH)], osem)
    for fb in range(4):
        pltpu.make_async_copy(
            outv.at[pl.ds(fb * 8, 8)],
            out_ref.at[0, 0, :, pl.ds(0, CH)], osem).wait()


_MESH = plsc.VectorSubcoreMesh(
    core_axis_name="c", subcore_axis_name="s", num_cores=NSC,
    num_subcores=NSUB)
_CPARAMS = pltpu.CompilerParams(
    needs_layout_passes=False, use_tc_tiling_on_sc=False)

_encode = functools.partial(
    pl.kernel,
    out_type=jax.ShapeDtypeStruct((4, NPTS // 128, 8, 128), jnp.float32),
    mesh=_MESH,
    compiler_params=_CPARAMS,
    scratch_types=[
        pltpu.VMEM((TABV_ROWS, PKW), jnp.float32),
        pltpu.VMEM((2 * 3 * CH,), jnp.float32),
        pltpu.VMEM((NSTR, 8 * CH), jnp.int32),
        pltpu.VMEM((NSTR * 8 * CH, PKW), jnp.float32),
        pltpu.VMEM((ODIM, CH), jnp.float32),
        pltpu.SemaphoreType.DMA,
        pltpu.SemaphoreType.DMA,
    ],
)(_body)


REP_IN_ROWS = NLV * (TBL // 128) * FEAT
REP_CH = 128
REP_PER_W = REP_IN_ROWS // NW


def _repack_body(tn_ref, pk_ref, inb, outb, sem):
    cid = lax.axis_index("c")
    sid = lax.axis_index("s")
    wid = sid * NSC + cid
    iota = lax.iota(jnp.int32, 16)
    r0 = wid * REP_PER_W

    @pl.loop(0, REP_PER_W // REP_CH)
    def _ck(ck):
        rin = r0 + ck * REP_CH
        pltpu.sync_copy(tn_ref.at[pl.ds(rin, REP_CH)], inb)

        @pl.loop(0, REP_CH // 2)
        def _pair(k):
            for g in range(8):
                o = g * 16
                f0 = inb[2 * k, pl.ds(o, 16)]
                f1 = inb[2 * k + 1, pl.ds(o, 16)]
                q0 = 2 * o + 2 * iota
                rows0 = k * 32 + (q0 >> 3)
                plsc.store_scatter(outb, [rows0, q0 & 7], f0)
                q1 = q0 + 1
                plsc.store_scatter(outb, [k * 32 + (q1 >> 3), q1 & 7], f1)

        pltpu.sync_copy(outb, pk_ref.at[pl.ds((rin // 2) * 32, REP_CH * 16)])


_repack = functools.partial(
    pl.kernel,
    out_type=jax.ShapeDtypeStruct((NLV * TBL // PK, PKW), jnp.float32),
    mesh=_MESH,
    compiler_params=_CPARAMS,
    scratch_types=[
        pltpu.VMEM((REP_CH, 128), jnp.float32),
        pltpu.VMEM((REP_CH * 16, PKW), jnp.float32),
        pltpu.SemaphoreType.DMA,
    ],
)(_repack_body)


def kernel(x, table):
    xt = x.T.reshape(-1)
    tabn = (table.reshape(NLV, TBL // 128, 128, FEAT)
            .transpose(0, 1, 3, 2)
            .reshape(REP_IN_ROWS, 128))
    tab4 = _repack(tabn)
    out4 = _encode(xt, tab4)
    return out4.transpose(1, 3, 0, 2).reshape(NPTS, ODIM)
